# baseline (speedup 1.0000x reference)
import jax
import jax.numpy as jnp
from jax import lax
from jax.experimental import pallas as pl
from jax.experimental.pallas import tpu as pltpu
from jax.experimental.pallas import tpu_sc as plsc

NC = 2
NS = 16
L = 16
NW = NC * NS

B = 4096
S = 50
M = 8
D = 32
C = B // NW
RC = C * M


def _stage(ids_hbm, mask_hbm, s, wb, bufid, bufmk, sem_in):
    pltpu.async_copy(ids_hbm.at[s, :, pl.ds(wb, C)], bufid, sem_in)
    pltpu.async_copy(mask_hbm.at[s, :, pl.ds(wb, C)], bufmk, sem_in)


def _stage_wait(ids_hbm, mask_hbm, s, wb, bufid, bufmk, sem_in):
    pltpu.make_async_copy(ids_hbm.at[s, :, pl.ds(wb, C)], bufid, sem_in).wait()
    pltpu.make_async_copy(mask_hbm.at[s, :, pl.ds(wb, C)], bufmk, sem_in).wait()


def _fire_gathers(table_hbm, bufid, rows, sem_g):
    for m in range(M):
        pltpu.async_copy(table_hbm.at[bufid.at[m]],
                         rows.at[pl.ds(m * C, C)], sem_g)


def _drain_gathers(table_hbm, bufid, rows, sem_g):
    for m in range(M):
        pltpu.make_async_copy(table_hbm.at[bufid.at[m]],
                              rows.at[pl.ds(m * C, C)], sem_g).wait()


def _fire_out(out_hbm, out_v, s, wb, sem_o):
    for d in range(D):
        pltpu.async_copy(out_v.at[:, pl.ds(d, 1)],
                         out_hbm.at[s, d, pl.ds(wb, C)], sem_o)


def _drain_out(out_hbm, out_v, wb, sem_o):
    for d in range(D):
        pltpu.make_async_copy(out_v.at[:, pl.ds(d, 1)],
                              out_hbm.at[0, d, pl.ds(wb, C)], sem_o).wait()


def _compute(bufmk, rows, out_v):
    def group_body(bg, c):
        bs = pl.ds(bg * L, L)
        mmf = [bufmk[m, bs].astype(jnp.float32) for m in range(M)]
        cnt = mmf[0]
        for m in range(1, M):
            cnt = cnt + mmf[m]
        a16 = 1.0 / jnp.maximum(cnt, 1.0)
        for t in range(L):
            b = bg * L + t
            lo = rows[b, pl.ds(0, L)] * mmf[0][t]
            hi = rows[b, pl.ds(L, L)] * mmf[0][t]
            for m in range(1, M):
                lo = lo + rows[m * C + b, pl.ds(0, L)] * mmf[m][t]
                hi = hi + rows[m * C + b, pl.ds(L, L)] * mmf[m][t]
            a = a16[t]
            out_v[b, pl.ds(0, L)] = lo * a
            out_v[b, pl.ds(L, L)] = hi * a
        return c
    lax.fori_loop(0, C // L, group_body, 0)


def _body(ids_hbm, mask_hbm, table_hbm, out_hbm,
          bufid0, bufid1, bufmk0, bufmk1, rows0, rows1, outv0, outv1,
          sem_in0, sem_in1, sem_g0, sem_g1, sem_o0, sem_o1):
    w = lax.axis_index("s") * NC + lax.axis_index("c")
    wb = w * C

    _stage(ids_hbm, mask_hbm, 0, wb, bufid0, bufmk0, sem_in0)
    _stage_wait(ids_hbm, mask_hbm, 0, wb, bufid0, bufmk0, sem_in0)
    _fire_gathers(table_hbm, bufid0, rows0, sem_g0)
    _stage(ids_hbm, mask_hbm, 1, wb, bufid1, bufmk1, sem_in1)

    def pair_body(g, carry):
        sa = 2 * g
        sb = sa + 1

        _stage_wait(ids_hbm, mask_hbm, sb, wb, bufid1, bufmk1, sem_in1)
        _fire_gathers(table_hbm, bufid1, rows1, sem_g1)

        _drain_gathers(table_hbm, bufid0, rows0, sem_g0)

        @pl.when(g > 0)
        def _():
            _drain_out(out_hbm, outv0, wb, sem_o0)
        _compute(bufmk0, rows0, outv0)
        _fire_out(out_hbm, outv0, sa, wb, sem_o0)

        @pl.when(g < S // 2 - 1)
        def _():
            _stage(ids_hbm, mask_hbm, sa + 2, wb, bufid0, bufmk0, sem_in0)
            _stage_wait(ids_hbm, mask_hbm, sa + 2, wb, bufid0, bufmk0, sem_in0)
            _fire_gathers(table_hbm, bufid0, rows0, sem_g0)

        _drain_gathers(table_hbm, bufid1, rows1, sem_g1)

        @pl.when(g > 0)
        def _():
            _drain_out(out_hbm, outv1, wb, sem_o1)
        _compute(bufmk1, rows1, outv1)
        _fire_out(out_hbm, outv1, sb, wb, sem_o1)

        @pl.when(g < S // 2 - 1)
        def _():
            _stage(ids_hbm, mask_hbm, sb + 2, wb, bufid1, bufmk1, sem_in1)
        return carry

    lax.fori_loop(0, S // 2, pair_body, 0)

    _drain_out(out_hbm, outv0, wb, sem_o0)
    _drain_out(out_hbm, outv1, wb, sem_o1)


def kernel(category_ids, category_mask, embedding_table):
    ids_t = jnp.transpose(category_ids.astype(jnp.int32), (1, 2, 0))
    mask_t = jnp.transpose(category_mask.astype(jnp.int32), (1, 2, 0))

    mesh = plsc.VectorSubcoreMesh(core_axis_name="c", subcore_axis_name="s",
                                  num_cores=NC, num_subcores=NS)
    out = pl.kernel(
        _body,
        out_type=jax.ShapeDtypeStruct((S, D, B, 1), jnp.float32),
        mesh=mesh,
        compiler_params=pltpu.CompilerParams(use_tc_tiling_on_sc=False),
        scratch_types=[
            pltpu.VMEM((M, C), jnp.int32),
            pltpu.VMEM((M, C), jnp.int32),
            pltpu.VMEM((M, C), jnp.int32),
            pltpu.VMEM((M, C), jnp.int32),
            pltpu.VMEM((RC, D), jnp.float32),
            pltpu.VMEM((RC, D), jnp.float32),
            pltpu.VMEM((C, D), jnp.float32),
            pltpu.VMEM((C, D), jnp.float32),
            pltpu.SemaphoreType.DMA,
            pltpu.SemaphoreType.DMA,
            pltpu.SemaphoreType.DMA,
            pltpu.SemaphoreType.DMA,
            pltpu.SemaphoreType.DMA,
            pltpu.SemaphoreType.DMA,
        ],
    )(ids_t, mask_t, embedding_table)
    return jnp.transpose(out.reshape(S, D, B), (2, 0, 1))


# speedup vs baseline: 21.8374x; 21.8374x over previous
import jax
import jax.numpy as jnp
from jax import lax
from jax.experimental import pallas as pl
from jax.experimental.pallas import tpu as pltpu
from jax.experimental.pallas import tpu_sc as plsc

NC = 2
NS = 16
L = 16
NW = NC * NS

B = 4096
S = 50
M = 8
D = 32
C = B // NW
RC = C * M


def _stage(ids_hbm, mask_hbm, s, wb, bufid, bufmk, sem_in):
    pltpu.async_copy(ids_hbm.at[s, :, pl.ds(wb, C)], bufid, sem_in)
    pltpu.async_copy(mask_hbm.at[s, :, pl.ds(wb, C)], bufmk, sem_in)


def _stage_wait(ids_hbm, mask_hbm, s, wb, bufid, bufmk, sem_in):
    pltpu.make_async_copy(ids_hbm.at[s, :, pl.ds(wb, C)], bufid, sem_in).wait()
    pltpu.make_async_copy(mask_hbm.at[s, :, pl.ds(wb, C)], bufmk, sem_in).wait()


def _fire_gathers(table_hbm, bufid, rows, sem_g):
    for m in range(M):
        pltpu.async_copy(table_hbm.at[bufid.at[m]],
                         rows.at[pl.ds(m * C, C)], sem_g)


def _drain_gathers(table_hbm, bufid, rows, sem_g):
    for m in range(M):
        pltpu.make_async_copy(table_hbm.at[bufid.at[m]],
                              rows.at[pl.ds(m * C, C)], sem_g).wait()


def _compute(bufmk, rows, out_v):
    def group_body(bg, c):
        bs = pl.ds(bg * L, L)
        mmf = [bufmk[m, bs].astype(jnp.float32) for m in range(M)]
        cnt = mmf[0]
        for m in range(1, M):
            cnt = cnt + mmf[m]
        a16 = 1.0 / jnp.maximum(cnt, 1.0)
        for t in range(L):
            b = bg * L + t
            lo = rows[b, pl.ds(0, L)] * mmf[0][t]
            hi = rows[b, pl.ds(L, L)] * mmf[0][t]
            for m in range(1, M):
                lo = lo + rows[m * C + b, pl.ds(0, L)] * mmf[m][t]
                hi = hi + rows[m * C + b, pl.ds(L, L)] * mmf[m][t]
            a = a16[t]
            out_v[b, pl.ds(0, L)] = lo * a
            out_v[b, pl.ds(L, L)] = hi * a
        return c
    lax.fori_loop(0, C // L, group_body, 0)


def _body(ids_hbm, mask_hbm, table_hbm, out_hbm,
          bufid0, bufid1, bufmk0, bufmk1, rows0, rows1, outv0, outv1,
          sem_in0, sem_in1, sem_g0, sem_g1, sem_o0, sem_o1):
    w = lax.axis_index("s") * NC + lax.axis_index("c")
    wb = w * C

    _stage(ids_hbm, mask_hbm, 0, wb, bufid0, bufmk0, sem_in0)
    _stage_wait(ids_hbm, mask_hbm, 0, wb, bufid0, bufmk0, sem_in0)
    _fire_gathers(table_hbm, bufid0, rows0, sem_g0)
    _stage(ids_hbm, mask_hbm, 1, wb, bufid1, bufmk1, sem_in1)

    def pair_body(g, carry):
        sa = 2 * g
        sb = sa + 1

        _stage_wait(ids_hbm, mask_hbm, sb, wb, bufid1, bufmk1, sem_in1)
        _fire_gathers(table_hbm, bufid1, rows1, sem_g1)

        _drain_gathers(table_hbm, bufid0, rows0, sem_g0)

        @pl.when(g > 0)
        def _():
            pltpu.make_async_copy(outv0, out_hbm.at[pl.ds(wb, C), 0],
                                  sem_o0).wait()
        _compute(bufmk0, rows0, outv0)
        pltpu.async_copy(outv0, out_hbm.at[pl.ds(wb, C), sa], sem_o0)

        @pl.when(g < S // 2 - 1)
        def _():
            _stage(ids_hbm, mask_hbm, sa + 2, wb, bufid0, bufmk0, sem_in0)
            _stage_wait(ids_hbm, mask_hbm, sa + 2, wb, bufid0, bufmk0, sem_in0)
            _fire_gathers(table_hbm, bufid0, rows0, sem_g0)

        _drain_gathers(table_hbm, bufid1, rows1, sem_g1)

        @pl.when(g > 0)
        def _():
            pltpu.make_async_copy(outv1, out_hbm.at[pl.ds(wb, C), 0],
                                  sem_o1).wait()
        _compute(bufmk1, rows1, outv1)
        pltpu.async_copy(outv1, out_hbm.at[pl.ds(wb, C), sb], sem_o1)

        @pl.when(g < S // 2 - 1)
        def _():
            _stage(ids_hbm, mask_hbm, sb + 2, wb, bufid1, bufmk1, sem_in1)
        return carry

    lax.fori_loop(0, S // 2, pair_body, 0)

    pltpu.make_async_copy(outv0, out_hbm.at[pl.ds(wb, C), 0], sem_o0).wait()
    pltpu.make_async_copy(outv1, out_hbm.at[pl.ds(wb, C), 0], sem_o1).wait()


def kernel(category_ids, category_mask, embedding_table):
    ids_t = jnp.transpose(category_ids.astype(jnp.int32), (1, 2, 0))
    mask_t = jnp.transpose(category_mask.astype(jnp.int32), (1, 2, 0))

    mesh = plsc.VectorSubcoreMesh(core_axis_name="c", subcore_axis_name="s",
                                  num_cores=NC, num_subcores=NS)
    out = pl.kernel(
        _body,
        out_type=jax.ShapeDtypeStruct((B, S, D), jnp.float32),
        mesh=mesh,
        compiler_params=pltpu.CompilerParams(use_tc_tiling_on_sc=False),
        scratch_types=[
            pltpu.VMEM((M, C), jnp.int32),
            pltpu.VMEM((M, C), jnp.int32),
            pltpu.VMEM((M, C), jnp.int32),
            pltpu.VMEM((M, C), jnp.int32),
            pltpu.VMEM((RC, D), jnp.float32),
            pltpu.VMEM((RC, D), jnp.float32),
            pltpu.VMEM((C, D), jnp.float32),
            pltpu.VMEM((C, D), jnp.float32),
            pltpu.SemaphoreType.DMA,
            pltpu.SemaphoreType.DMA,
            pltpu.SemaphoreType.DMA,
            pltpu.SemaphoreType.DMA,
            pltpu.SemaphoreType.DMA,
            pltpu.SemaphoreType.DMA,
        ],
    )(ids_t, mask_t, embedding_table)
    return out
